# Initial kernel scaffold; baseline (speedup 1.0000x reference)
import jax
import jax.numpy as jnp
from jax.experimental import pallas as pl


def kernel(hidden_states, router_weight):
    raise NotImplementedError("write your pallas kernel here")


# speedup vs baseline: 1.0009x; 1.0009x over previous
import functools

import jax
import jax.numpy as jnp
from jax import lax
from jax.experimental import pallas as pl
from jax.experimental.pallas import tpu as pltpu
from jax.experimental.pallas import tpu_sc as plsc

_E = 64
_D = 4096
_L = 16
_NW = 32


def _logits_body(x_ref, w_ref, out_ref):
    out_ref[...] = lax.dot_general(
        x_ref[...], w_ref[...],
        dimension_numbers=(((1,), (1,)), ((), ())),
        preferred_element_type=jnp.float32)


def _router_logits(x, w, blk):
    t = x.shape[0]
    return pl.pallas_call(
        _logits_body,
        grid=(t // blk,),
        in_specs=[
            pl.BlockSpec((blk, _D), lambda i: (i, 0)),
            pl.BlockSpec((_E, _D), lambda i: (0, 0)),
        ],
        out_specs=pl.BlockSpec((blk, _E), lambda i: (i, 0)),
        out_shape=jax.ShapeDtypeStruct((t, _E), jnp.float32),
    )(x, w)


def _make_router(t):
    tok_w = t // _NW
    mesh = plsc.VectorSubcoreMesh(core_axis_name="c", subcore_axis_name="s")

    @functools.partial(
        pl.kernel,
        mesh=mesh,
        out_type=[jax.ShapeDtypeStruct((t * 2,), jnp.float32),
                  jax.ShapeDtypeStruct((t * 2,), jnp.int32)],
        scratch_types=[pltpu.VMEM((tok_w * _E,), jnp.float32),
                       pltpu.VMEM((tok_w * 2,), jnp.float32),
                       pltpu.VMEM((tok_w * 2,), jnp.int32)],
        compiler_params=pltpu.CompilerParams(needs_layout_passes=False),
    )
    def route(logits_hbm, w_hbm, i_hbm, buf, wbuf, ibuf):
        wid = lax.axis_index("s") * 2 + lax.axis_index("c")
        base = wid * tok_w
        pltpu.sync_copy(logits_hbm.at[pl.ds(base * _E, tok_w * _E)], buf)
        lanes = lax.iota(jnp.int32, _L)

        def group(g, carry):
            row = g * _L + lanes
            neg = jnp.full((_L,), -3.0e38, jnp.float32)
            zero = jnp.zeros((_L,), jnp.int32)

            def expert(j, c):
                m1, i1, m2, i2 = c
                col = jnp.full((_L,), j, jnp.int32)
                v = plsc.load_gather(buf, [row * _E + col])
                gt1 = v > m1
                gt2 = v > m2
                nm2 = jnp.where(gt1, m1, jnp.where(gt2, v, m2))
                ni2 = jnp.where(gt1, i1, jnp.where(gt2, col, i2))
                nm1 = jnp.where(gt1, v, m1)
                ni1 = jnp.where(gt1, col, i1)
                return nm1, ni1, nm2, ni2

            m1, i1, m2, i2 = lax.fori_loop(
                0, _E, expert, (neg, zero, neg, zero))
            e2 = jnp.exp(m2 - m1)
            w1 = 1.0 / (1.0 + e2)
            w2 = 1.0 - w1
            plsc.store_scatter(wbuf, [row * 2], w1)
            plsc.store_scatter(wbuf, [row * 2 + 1], w2)
            plsc.store_scatter(ibuf, [row * 2], i1)
            plsc.store_scatter(ibuf, [row * 2 + 1], i2)
            return carry

        lax.fori_loop(0, tok_w // _L, group, 0)
        pltpu.sync_copy(wbuf, w_hbm.at[pl.ds(base * 2, tok_w * 2)])
        pltpu.sync_copy(ibuf, i_hbm.at[pl.ds(base * 2, tok_w * 2)])

    return route


def kernel(hidden_states, router_weight):
    b, s, d = hidden_states.shape
    t = b * s
    x = hidden_states.reshape(t, d)
    logits = _router_logits(x, router_weight, blk=512)
    w, idx = _make_router(t)(logits.reshape(t * _E))
    return (w.reshape(b, s, 2), idx.reshape(b, s, 2),
            logits.reshape(b, s, _E))
